# baseline (device time: 318178 ns/iter reference)
import jax
import jax.numpy as jnp
from jax import lax
from jax.experimental import pallas as pl
from jax.experimental.pallas import tpu as pltpu

N_DEV = 32
B, SQ, D = 4, 256, 1024
HQ_LOC, HKV_LOC, DH = 8, 2, 128
SKV = 1024
SCALE = 0.08838834764831843
ROWS = B * SQ
CHUNK = ROWS // N_DEV


def _body(x_ref, wq_ref, wo_ref, kt_ref, vt_ref, out_ref,
          q_ref, rbuf, send_sems, recv_sems, credit_sem):
    i = lax.axis_index("i")
    left = jnp.mod(i - 1, N_DEV)
    right = jnp.mod(i + 1, N_DEV)

    barrier = pltpu.get_barrier_semaphore()
    for nbr in (left, right):
        pl.semaphore_signal(barrier, inc=1, device_id=(nbr,),
                            device_id_type=pl.DeviceIdType.MESH)
    pl.semaphore_wait(barrier, 2)

    q = jnp.dot(x_ref[...], wq_ref[...], preferred_element_type=jnp.float32)
    q_ref[...] = (q * SCALE).astype(jnp.bfloat16)

    for b in range(B):
        rows = slice(b * SQ, (b + 1) * SQ)
        acc = None
        for g in range(HKV_LOC):
            k_bg = kt_ref[b, g, :, :]
            v_bg = vt_ref[b, g, :, :]
            for hh in range(HQ_LOC // HKV_LOC):
                h = g * (HQ_LOC // HKV_LOC) + hh
                q_bh = q_ref[rows, h * DH:(h + 1) * DH]
                s = jnp.dot(q_bh, k_bg, preferred_element_type=jnp.float32)
                m = jnp.max(s, axis=1, keepdims=True)
                p = jnp.exp(s - m)
                l = jnp.sum(p, axis=1, keepdims=True)
                o = jnp.dot(p.astype(jnp.bfloat16), v_bg,
                            preferred_element_type=jnp.float32)
                o = (o / l).astype(jnp.bfloat16)
                contrib = jnp.dot(o, wo_ref[h * DH:(h + 1) * DH, :],
                                  preferred_element_type=jnp.float32)
                acc = contrib if acc is None else acc + contrib
        out_ref[rows, :] = acc

    n_hops = 2 * (N_DEV - 1)
    for hop in range(n_hops):
        if hop < N_DEV - 1:
            sc = jnp.mod(i - hop, N_DEV)
            src = out_ref.at[pl.ds(sc * CHUNK, CHUNK), :]
            dst = rbuf.at[hop % 2]
        else:
            s_ = hop - (N_DEV - 1)
            sc = jnp.mod(i + 1 - s_, N_DEV)
            src = out_ref.at[pl.ds(sc * CHUNK, CHUNK), :]
            dst = out_ref.at[pl.ds(sc * CHUNK, CHUNK), :]

        if hop >= 2:
            pl.semaphore_wait(credit_sem, 1)
        rdma = pltpu.make_async_remote_copy(
            src_ref=src, dst_ref=dst,
            send_sem=send_sems.at[hop % 2], recv_sem=recv_sems.at[hop % 2],
            device_id=(right,), device_id_type=pl.DeviceIdType.MESH,
        )
        rdma.start()
        rdma.wait()

        if hop < N_DEV - 1:
            rc = jnp.mod(i - hop - 1, N_DEV)
            rws = pl.ds(rc * CHUNK, CHUNK)
            out_ref[rws, :] = out_ref[rws, :] + rbuf[hop % 2]
        if hop < n_hops - 2:
            pl.semaphore_signal(credit_sem, inc=1, device_id=(left,),
                                device_id_type=pl.DeviceIdType.MESH)


def kernel(x, Wq, Wo, K_ext, V_ext):
    i = lax.axis_index("i")

    K_sl = lax.dynamic_slice_in_dim(K_ext, 2 * i, HKV_LOC, axis=2)
    V_sl = lax.dynamic_slice_in_dim(V_ext, 2 * i, HKV_LOC, axis=2)
    kt = jnp.transpose(K_sl, (0, 2, 3, 1)).astype(jnp.bfloat16)
    vt = jnp.transpose(V_sl, (0, 2, 1, 3)).astype(jnp.bfloat16)
    x2 = x.reshape(ROWS, D).astype(jnp.bfloat16)
    wq = Wq.astype(jnp.bfloat16)
    wo = Wo.astype(jnp.bfloat16)

    out = pl.pallas_call(
        _body,
        out_shape=jax.ShapeDtypeStruct((ROWS, D), jnp.float32),
        in_specs=[pl.BlockSpec(memory_space=pltpu.VMEM)] * 5,
        out_specs=pl.BlockSpec(memory_space=pltpu.VMEM),
        scratch_shapes=[
            pltpu.VMEM((ROWS, HQ_LOC * DH), jnp.bfloat16),
            pltpu.VMEM((2, CHUNK, D), jnp.float32),
            pltpu.SemaphoreType.DMA((2,)),
            pltpu.SemaphoreType.DMA((2,)),
            pltpu.SemaphoreType.REGULAR,
        ],
        compiler_params=pltpu.CompilerParams(collective_id=0),
    )(x2, wq, wo, kt, vt)
    return out.reshape(B, SQ, D)


# device time: 177949 ns/iter; 1.7880x vs baseline; 1.7880x over previous
import functools

import jax
import jax.numpy as jnp
from jax import lax
from jax.experimental import pallas as pl
from jax.experimental.pallas import tpu as pltpu

N_DEV = 32
B, SQ, D = 4, 256, 1024
HQ_LOC, HKV_LOC, DH = 8, 2, 128
SKV = 1024
SCALE = 0.08838834764831843
ROWS = B * SQ
HALF = ROWS // 2
QCH = HALF // 4
ZCH = QCH // 4

_MESH = pl.DeviceIdType.MESH


def _logi(xx, yy, zz):
    return zz * 8 + yy * 2 + jnp.where(yy % 2 == 0, xx, 1 - xx)


def _body(x_ref, wq_ref, wo_ref, kt_ref, vt_ref, out_ref,
          q_ref, rbx, rby, rbz, send_sems, recv_sems):
    lid = lax.axis_index("i")
    z = lid // 8
    r = lid % 8
    y = r // 2
    xb = jnp.where(y % 2 == 0, r % 2, 1 - (r % 2))

    px = _logi(1 - xb, y, z)
    y_r = _logi(xb, (y + 1) % 4, z)
    y_l = _logi(xb, (y - 1) % 4, z)
    z_r = (lid + 8) % N_DEV
    z_l = (lid - 8) % N_DEV
    partners = (px, y_l, y_r, z_l, z_r)

    barrier = pltpu.get_barrier_semaphore()
    for p in partners:
        pl.semaphore_signal(barrier, inc=1, device_id=(p,),
                            device_id_type=_MESH)
    pl.semaphore_wait(barrier, len(partners))

    q = jnp.dot(x_ref[...], wq_ref[...], preferred_element_type=jnp.float32)
    q_ref[...] = (q * SCALE).astype(jnp.bfloat16)

    for b in range(B):
        rows = slice(b * SQ, (b + 1) * SQ)
        acc = None
        for g in range(HKV_LOC):
            k_bg = kt_ref[b, g, :, :]
            v_bg = vt_ref[b, g, :, :]
            for hh in range(HQ_LOC // HKV_LOC):
                h = g * (HQ_LOC // HKV_LOC) + hh
                q_bh = q_ref[rows, h * DH:(h + 1) * DH]
                s = jnp.dot(q_bh, k_bg, preferred_element_type=jnp.float32)
                m = jnp.max(s, axis=1, keepdims=True)
                p = jnp.exp(s - m)
                l = jnp.sum(p, axis=1, keepdims=True)
                o = jnp.dot(p.astype(jnp.bfloat16), v_bg,
                            preferred_element_type=jnp.float32)
                o = (o / l).astype(jnp.bfloat16)
                contrib = jnp.dot(o, wo_ref[h * DH:(h + 1) * DH, :],
                                  preferred_element_type=jnp.float32)
                acc = contrib if acc is None else acc + contrib
        out_ref[rows, :] = acc

    def step(sem_idx, src, dst, target, add_to=None, add_from=None):
        rdma = pltpu.make_async_remote_copy(
            src_ref=src, dst_ref=dst,
            send_sem=send_sems.at[sem_idx], recv_sem=recv_sems.at[sem_idx],
            device_id=(target,), device_id_type=_MESH,
        )
        rdma.start()
        rdma.wait()
        if add_to is not None:
            out_ref[add_to, :] = out_ref[add_to, :] + add_from()

    hm = xb * HALF
    hp = (1 - xb) * HALF

    step(0, out_ref.at[pl.ds(hp, HALF), :], rbx, px,
         add_to=pl.ds(hm, HALF), add_from=lambda: rbx[:, :])

    for t in range(3):
        qs = (y - t) % 4
        qr = (y - t - 1) % 4
        step(1 + t, out_ref.at[pl.ds(hm + qs * QCH, QCH), :], rby.at[t], y_r,
             add_to=pl.ds(hm + qr * QCH, QCH), add_from=lambda t=t: rby[t, :, :])
    base1 = hm + ((y + 1) % 4) * QCH

    for t in range(3):
        cs = (z - t) % 4
        cr = (z - t - 1) % 4
        step(4 + t, out_ref.at[pl.ds(base1 + cs * ZCH, ZCH), :], rbz.at[t],
             z_r, add_to=pl.ds(base1 + cr * ZCH, ZCH),
             add_from=lambda t=t: rbz[t, :, :])

    for t in range(3):
        cc = (z + 1 - t) % 4
        sl = out_ref.at[pl.ds(base1 + cc * ZCH, ZCH), :]
        step(7 + t, sl, sl, z_r)

    for t in range(3):
        qq = (y + 1 - t) % 4
        sl = out_ref.at[pl.ds(hm + qq * QCH, QCH), :]
        step(10 + t, sl, sl, y_r)

    sl = out_ref.at[pl.ds(hm, HALF), :]
    step(13, sl, sl, px)

    @functools.partial(pl.run_scoped, sb=pltpu.SemaphoreType.REGULAR)
    def _(sb):
        for p in partners:
            pl.semaphore_signal(sb, inc=1, device_id=(p,),
                                device_id_type=_MESH)
        pl.semaphore_wait(sb, len(partners))


def kernel(x, Wq, Wo, K_ext, V_ext):
    i = lax.axis_index("i")

    K_sl = lax.dynamic_slice_in_dim(K_ext, 2 * i, HKV_LOC, axis=2)
    V_sl = lax.dynamic_slice_in_dim(V_ext, 2 * i, HKV_LOC, axis=2)
    kt = jnp.transpose(K_sl, (0, 2, 3, 1)).astype(jnp.bfloat16)
    vt = jnp.transpose(V_sl, (0, 2, 1, 3)).astype(jnp.bfloat16)
    x2 = x.reshape(ROWS, D).astype(jnp.bfloat16)
    wq = Wq.astype(jnp.bfloat16)
    wo = Wo.astype(jnp.bfloat16)

    out = pl.pallas_call(
        _body,
        out_shape=jax.ShapeDtypeStruct((ROWS, D), jnp.float32),
        in_specs=[pl.BlockSpec(memory_space=pltpu.VMEM)] * 5,
        out_specs=pl.BlockSpec(memory_space=pltpu.VMEM),
        scratch_shapes=[
            pltpu.VMEM((ROWS, HQ_LOC * DH), jnp.bfloat16),
            pltpu.VMEM((HALF, D), jnp.float32),
            pltpu.VMEM((3, QCH, D), jnp.float32),
            pltpu.VMEM((3, ZCH, D), jnp.float32),
            pltpu.SemaphoreType.DMA((14,)),
            pltpu.SemaphoreType.DMA((14,)),
        ],
        compiler_params=pltpu.CompilerParams(collective_id=0),
    )(x2, wq, wo, kt, vt)
    return out.reshape(B, SQ, D)


# device time: 130533 ns/iter; 2.4375x vs baseline; 1.3632x over previous
import functools

import jax
import jax.numpy as jnp
from jax import lax
from jax.experimental import pallas as pl
from jax.experimental.pallas import tpu as pltpu

N_DEV = 32
B, SQ, D = 4, 256, 1024
HQ_LOC, HKV_LOC, DH = 8, 2, 128
SKV = 1024
SCALE = 0.08838834764831843
ROWS = B * SQ
HALF = ROWS // 2
QCH = HALF // 4
ZCH = QCH // 4

_MESH = pl.DeviceIdType.MESH

_RSX = 0
_RSY = 4
_RSZ = 7
_AGZ = 10
_AGY = 13
_AGX = 16
_NSEM = 20


def _logi(xx, yy, zz):
    return zz * 8 + yy * 2 + jnp.where(yy % 2 == 0, xx, 1 - xx)


def _body(x_ref, wq_ref, wo_ref, kt_ref, vt_ref, out_ref,
          q_ref, rbx, rby, rbz, abuf, send_sems, recv_sems):
    lid = lax.axis_index("i")
    z = lid // 8
    r = lid % 8
    y = r // 2
    xb = jnp.where(y % 2 == 0, r % 2, 1 - (r % 2))

    px = _logi(1 - xb, y, z)
    y_r = _logi(xb, (y + 1) % 4, z)
    y_l = _logi(xb, (y - 1) % 4, z)
    z_r = (lid + 8) % N_DEV
    z_l = (lid - 8) % N_DEV
    partners = (px, y_l, y_r, z_l, z_r)

    barrier = pltpu.get_barrier_semaphore()
    for p in partners:
        pl.semaphore_signal(barrier, inc=1, device_id=(p,),
                            device_id_type=_MESH)
    pl.semaphore_wait(barrier, len(partners))

    q = jnp.dot(x_ref[...], wq_ref[...], preferred_element_type=jnp.float32)
    q_ref[...] = (q * SCALE).astype(jnp.bfloat16)

    for b in range(B):
        rows = slice(b * SQ, (b + 1) * SQ)
        acc = None
        for g in range(HKV_LOC):
            k_bg = kt_ref[b, g, :, :]
            v_bg = vt_ref[b, g, :, :]
            for hh in range(HQ_LOC // HKV_LOC):
                h = g * (HQ_LOC // HKV_LOC) + hh
                q_bh = q_ref[rows, h * DH:(h + 1) * DH]
                s = jnp.dot(q_bh, k_bg, preferred_element_type=jnp.float32)
                m = jnp.max(s, axis=1, keepdims=True)
                p = jnp.exp(s - m)
                l = jnp.sum(p, axis=1, keepdims=True)
                o = jnp.dot(p.astype(jnp.bfloat16), v_bg,
                            preferred_element_type=jnp.float32)
                o = (o / l).astype(jnp.bfloat16)
                contrib = jnp.dot(o, wo_ref[h * DH:(h + 1) * DH, :],
                                  preferred_element_type=jnp.float32)
                acc = contrib if acc is None else acc + contrib
        out_ref[rows, :] = acc

    def rdma(sem_idx, src, dst, target):
        return pltpu.make_async_remote_copy(
            src_ref=src, dst_ref=dst,
            send_sem=send_sems.at[sem_idx], recv_sem=recv_sems.at[sem_idx],
            device_id=(target,), device_id_type=_MESH,
        )

    hm = xb * HALF
    hp = (1 - xb) * HALF

    xrd = []
    for t in range(4):
        c = (y - t) % 4
        rd = rdma(_RSX + t,
                  out_ref.at[pl.ds(hp + c * QCH, QCH), :],
                  rbx.at[pl.ds(c * QCH, QCH), :], px)
        rd.start()
        xrd.append(rd)

    for t in range(4):
        xrd[t].wait()
        c = (y - t) % 4
        out_ref[pl.ds(hm + c * QCH, QCH), :] = (
            out_ref[pl.ds(hm + c * QCH, QCH), :]
            + rbx[pl.ds(c * QCH, QCH), :])
        if t < 3:
            qr = (y - t - 1) % 4
            rd = rdma(_RSY + t,
                      out_ref.at[pl.ds(hm + c * QCH, QCH), :],
                      rby.at[t], y_r)
            rd.start()
            rd.wait()
            out_ref[pl.ds(hm + qr * QCH, QCH), :] = (
                out_ref[pl.ds(hm + qr * QCH, QCH), :] + rby[t, :, :])
    base1 = hm + ((y + 1) % 4) * QCH

    for t in range(3):
        cs = (z - t) % 4
        cr = (z - t - 1) % 4
        rd = rdma(_RSZ + t,
                  out_ref.at[pl.ds(base1 + cs * ZCH, ZCH), :],
                  rbz.at[t], z_r)
        rd.start()
        rd.wait()
        out_ref[pl.ds(base1 + cr * ZCH, ZCH), :] = (
            out_ref[pl.ds(base1 + cr * ZCH, ZCH), :] + rbz[t, :, :])

    bf = base1 + ((z + 1) % 4) * ZCH
    abuf[pl.ds(bf, ZCH), :] = out_ref[pl.ds(bf, ZCH), :].astype(jnp.bfloat16)

    for t in range(3):
        cc = (z + 1 - t) % 4
        sl = abuf.at[pl.ds(base1 + cc * ZCH, ZCH), :]
        rd = rdma(_AGZ + t, sl, sl, z_r)
        rd.start()
        rd.wait()

    agx = []
    for t in range(3):
        blk = (y + 1 - t) % 4
        sl = abuf.at[pl.ds(hm + blk * QCH, QCH), :]
        rdx = rdma(_AGX + t, sl, sl, px)
        rdx.start()
        agx.append(rdx)
        rdy = rdma(_AGY + t, sl, sl, y_r)
        rdy.start()
        rdy.wait()
    blk3 = (y - 2) % 4
    sl = abuf.at[pl.ds(hm + blk3 * QCH, QCH), :]
    rdx = rdma(_AGX + 3, sl, sl, px)
    rdx.start()
    agx.append(rdx)
    for rd in agx:
        rd.wait()

    out_ref[...] = abuf[...].astype(jnp.float32)

    @functools.partial(pl.run_scoped, sb=pltpu.SemaphoreType.REGULAR)
    def _(sb):
        for p in partners:
            pl.semaphore_signal(sb, inc=1, device_id=(p,),
                                device_id_type=_MESH)
        pl.semaphore_wait(sb, len(partners))


def kernel(x, Wq, Wo, K_ext, V_ext):
    i = lax.axis_index("i")

    K_sl = lax.dynamic_slice_in_dim(K_ext, 2 * i, HKV_LOC, axis=2)
    V_sl = lax.dynamic_slice_in_dim(V_ext, 2 * i, HKV_LOC, axis=2)
    kt = jnp.transpose(K_sl, (0, 2, 3, 1)).astype(jnp.bfloat16)
    vt = jnp.transpose(V_sl, (0, 2, 1, 3)).astype(jnp.bfloat16)
    x2 = x.reshape(ROWS, D).astype(jnp.bfloat16)
    wq = Wq.astype(jnp.bfloat16)
    wo = Wo.astype(jnp.bfloat16)

    out = pl.pallas_call(
        _body,
        out_shape=jax.ShapeDtypeStruct((ROWS, D), jnp.float32),
        in_specs=[pl.BlockSpec(memory_space=pltpu.VMEM)] * 5,
        out_specs=pl.BlockSpec(memory_space=pltpu.VMEM),
        scratch_shapes=[
            pltpu.VMEM((ROWS, HQ_LOC * DH), jnp.bfloat16),
            pltpu.VMEM((HALF, D), jnp.float32),
            pltpu.VMEM((3, QCH, D), jnp.float32),
            pltpu.VMEM((3, ZCH, D), jnp.float32),
            pltpu.VMEM((ROWS, D), jnp.bfloat16),
            pltpu.SemaphoreType.DMA((_NSEM,)),
            pltpu.SemaphoreType.DMA((_NSEM,)),
        ],
        compiler_params=pltpu.CompilerParams(collective_id=0),
    )(x2, wq, wo, kt, vt)
    return out.reshape(B, SQ, D)
